# Initial kernel scaffold; baseline (speedup 1.0000x reference)
import jax
import jax.numpy as jnp
from jax.experimental import pallas as pl


def kernel(thresholds, rules, binary_ops, left, right, leaf_is_reroll, leaf_mask_left, leaf_mask_right, leaf_mask_op, leaf_score_cat):
    raise NotImplementedError("write your pallas kernel here")


# speedup vs baseline: 11104.3456x; 11104.3456x over previous
import functools

import jax
import jax.numpy as jnp
from jax import lax
from jax.experimental import pallas as pl
from jax.experimental.pallas import tpu as pltpu
from jax.experimental.pallas import tpu_sc as plsc

N = 8192
LANES = 16
NCH = N // LANES


def _sc_body(left_hbm, right_hbm, reroll_hbm,
             mask_out, score_out, reroll_out, cnt_out,
             left_v, right_v, aux_v, buf2_v, work_v, cnt_v):
    c = lax.axis_index("c")
    s = lax.axis_index("s")

    @pl.when((s == 0) & (c == 0))
    def _reachability():
        pltpu.sync_copy(left_hbm, left_v)
        pltpu.sync_copy(right_hbm, right_v)

        def zero_chunk(i, carry):
            work_v[pl.ds(i * LANES, LANES)] = jnp.zeros((LANES,), jnp.int32)
            return carry

        lax.fori_loop(0, NCH, zero_chunk, 0)
        onehot0 = (lax.broadcasted_iota(jnp.int32, (LANES,), 0) == 0)
        work_v[pl.ds(0, LANES)] = onehot0.astype(jnp.int32)

        ones = jnp.ones((LANES,), jnp.int32)

        def one_pass(carry):
            _, cur = carry

            def chunk(i, acc):
                base = i * LANES
                v = work_v[pl.ds(base, LANES)]
                lv = left_v[pl.ds(base, LANES)]
                rv = right_v[pl.ds(base, LANES)]
                reach = v != 0
                ml = reach & (lv >= 0)
                mr = reach & (rv >= 0)
                il = jnp.where(ml, lv, 0)
                ir = jnp.where(mr, rv, 0)
                plsc.store_scatter(work_v, [il], ones, mask=ml)
                plsc.store_scatter(work_v, [ir], ones, mask=mr)
                return acc + v

            acc = lax.fori_loop(0, NCH, chunk, jnp.zeros((LANES,), jnp.int32))
            return (cur, jnp.sum(acc))

        _, final_cnt = lax.while_loop(
            lambda pc: pc[0] != pc[1], one_pass,
            (jnp.int32(-1), jnp.int32(0)))

        cnt_v[...] = jnp.full((LANES,), final_cnt, jnp.int32)
        pltpu.sync_copy(work_v, mask_out)
        pltpu.sync_copy(cnt_v, cnt_out)

    @pl.when((s == 0) & (c == 1))
    def _leaf_flags():
        pltpu.sync_copy(left_hbm, left_v)
        pltpu.sync_copy(right_hbm, right_v)
        pltpu.sync_copy(reroll_hbm, aux_v)

        def init_chunk(i, carry):
            base = i * LANES
            lv = left_v[pl.ds(base, LANES)]
            rv = right_v[pl.ds(base, LANES)]
            ml = lv < 0
            mr = rv < 0
            lid = jnp.where(ml, -lv - 1, 0)
            rid = jnp.where(mr, -rv - 1, 0)
            lr = plsc.load_gather(aux_v, [lid], mask=ml)
            rr = plsc.load_gather(aux_v, [rid], mask=mr)
            lr = jnp.where(ml, lr, 0)
            rr = jnp.where(mr, rr, 0)
            score = (ml & (lr == 0)) | (mr & (rr == 0))
            reroll = (ml & (lr != 0)) | (mr & (rr != 0))
            work_v[pl.ds(base, LANES)] = (
                score.astype(jnp.int32) | (reroll.astype(jnp.int32) << 1))
            return carry

        lax.fori_loop(0, NCH, init_chunk, 0)

        def sweep(t, carry):
            i = NCH - 1 - t
            base = i * LANES
            nid = base + lax.broadcasted_iota(jnp.int32, (LANES,), 0)
            lv = left_v[pl.ds(base, LANES)]
            rv = right_v[pl.ds(base, LANES)]
            ml = lv > nid
            mr = rv > nid
            il = jnp.where(ml, lv, 0)
            ir = jnp.where(mr, rv, 0)

            def upd(_):
                v = work_v[pl.ds(base, LANES)]
                gl = plsc.load_gather(work_v, [il], mask=ml)
                gr = plsc.load_gather(work_v, [ir], mask=mr)
                gl = jnp.where(ml, gl, 0)
                gr = jnp.where(mr, gr, 0)
                nv = v | gl | gr
                work_v[pl.ds(base, LANES)] = nv
                return jnp.any(nv != v)

            lax.while_loop(lambda ch: ch, upd, jnp.bool_(True))
            return carry

        lax.fori_loop(0, NCH, sweep, 0)

        def unpack_chunk(i, carry):
            base = i * LANES
            v = work_v[pl.ds(base, LANES)]
            aux_v[pl.ds(base, LANES)] = v & 1
            buf2_v[pl.ds(base, LANES)] = (v >> 1) & 1
            return carry

        lax.fori_loop(0, NCH, unpack_chunk, 0)
        pltpu.sync_copy(aux_v, score_out)
        pltpu.sync_copy(buf2_v, reroll_out)


@jax.jit
def _dag_flags(left, right, reroll_i32):
    mesh = plsc.VectorSubcoreMesh(core_axis_name="c", subcore_axis_name="s")
    f = pl.kernel(
        _sc_body,
        out_type=(
            jax.ShapeDtypeStruct((N,), jnp.int32),
            jax.ShapeDtypeStruct((N,), jnp.int32),
            jax.ShapeDtypeStruct((N,), jnp.int32),
            jax.ShapeDtypeStruct((LANES,), jnp.int32),
        ),
        mesh=mesh,
        compiler_params=pltpu.CompilerParams(needs_layout_passes=False),
        scratch_types=(
            pltpu.VMEM((N,), jnp.int32),
            pltpu.VMEM((N,), jnp.int32),
            pltpu.VMEM((N,), jnp.int32),
            pltpu.VMEM((N,), jnp.int32),
            pltpu.VMEM((N,), jnp.int32),
            pltpu.VMEM((LANES,), jnp.int32),
        ),
    )
    return f(left, right, reroll_i32)


def kernel(thresholds, rules, binary_ops, left, right, leaf_is_reroll,
           leaf_mask_left, leaf_mask_right, leaf_mask_op, leaf_score_cat):
    mask_i, score_i, reroll_i, cnt = _dag_flags(
        left, right, leaf_is_reroll.astype(jnp.int32))
    return (mask_i.astype(jnp.bool_), score_i.astype(jnp.bool_),
            reroll_i.astype(jnp.bool_), cnt[0])


# speedup vs baseline: 16440.4462x; 1.4805x over previous
import functools

import jax
import jax.numpy as jnp
from jax import lax
from jax.experimental import pallas as pl
from jax.experimental.pallas import tpu as pltpu
from jax.experimental.pallas import tpu_sc as plsc

N = 8192
LANES = 16
NCH = N // LANES


def _sc_body(left_hbm, right_hbm, reroll_hbm,
             mask_out, score_out, reroll_out, cnt_out,
             left_v, right_v, reroll_v, aux_v, buf2_v, work_v, cnt_v):
    c = lax.axis_index("c")
    s = lax.axis_index("s")

    @pl.when((s == 0) & (c == 0))
    def _reachability():
        pltpu.sync_copy(left_hbm, left_v)
        pltpu.sync_copy(right_hbm, right_v)

        def zero_chunk(i, carry):
            work_v[pl.ds(i * LANES, LANES)] = jnp.zeros((LANES,), jnp.int32)
            return carry

        lax.fori_loop(0, NCH, zero_chunk, 0)
        onehot0 = (lax.broadcasted_iota(jnp.int32, (LANES,), 0) == 0)
        work_v[pl.ds(0, LANES)] = onehot0.astype(jnp.int32)

        ones = jnp.ones((LANES,), jnp.int32)

        def one_pass(carry):
            _, cur = carry

            def chunk(i, acc):
                base = i * LANES
                v = work_v[pl.ds(base, LANES)]
                lv = left_v[pl.ds(base, LANES)]
                rv = right_v[pl.ds(base, LANES)]
                reach = v != 0
                ml = reach & (lv >= 0)
                mr = reach & (rv >= 0)
                il = jnp.where(ml, lv, 0)
                ir = jnp.where(mr, rv, 0)
                plsc.store_scatter(work_v, [il], ones, mask=ml)
                plsc.store_scatter(work_v, [ir], ones, mask=mr)
                return acc + v

            acc = lax.fori_loop(0, NCH, chunk, jnp.zeros((LANES,), jnp.int32))
            return (cur, jnp.sum(acc))

        _, final_cnt = lax.while_loop(
            lambda pc: pc[0] != pc[1], one_pass,
            (jnp.int32(-1), jnp.int32(0)))

        cnt_v[...] = jnp.full((LANES,), final_cnt, jnp.int32)
        pltpu.sync_copy(work_v, mask_out)
        pltpu.sync_copy(cnt_v, cnt_out)

    @pl.when((s == 1) & (c == 0))
    def _leaf_flags():
        pltpu.sync_copy(left_hbm, left_v)
        pltpu.sync_copy(right_hbm, right_v)
        pltpu.sync_copy(reroll_hbm, reroll_v)

        def sweep(t, carry):
            i = NCH - 1 - t
            base = i * LANES
            nid = base + lax.broadcasted_iota(jnp.int32, (LANES,), 0)
            lv = left_v[pl.ds(base, LANES)]
            rv = right_v[pl.ds(base, LANES)]

            mleaf_l = lv < 0
            mleaf_r = rv < 0
            lid = jnp.where(mleaf_l, -lv - 1, 0)
            rid = jnp.where(mleaf_r, -rv - 1, 0)
            lr = plsc.load_gather(reroll_v, [lid], mask=mleaf_l)
            rr = plsc.load_gather(reroll_v, [rid], mask=mleaf_r)
            lr = jnp.where(mleaf_l, lr, 0)
            rr = jnp.where(mleaf_r, rr, 0)
            score = (mleaf_l & (lr == 0)) | (mleaf_r & (rr == 0))
            reroll = (mleaf_l & (lr != 0)) | (mleaf_r & (rr != 0))
            basev = score.astype(jnp.int32) | (reroll.astype(jnp.int32) << 1)
            work_v[pl.ds(base, LANES)] = basev

            ml = lv > nid
            mr = rv > nid
            il = jnp.where(ml, lv, 0)
            ir = jnp.where(mr, rv, 0)
            gl = plsc.load_gather(work_v, [il], mask=ml)
            gr = plsc.load_gather(work_v, [ir], mask=mr)
            gl = jnp.where(ml, gl, 0)
            gr = jnp.where(mr, gr, 0)
            work_v[pl.ds(base, LANES)] = basev | gl | gr

            inchunk = (ml & (lv < base + LANES)) | (mr & (rv < base + LANES))

            @pl.when(jnp.any(inchunk))
            def _fixpoint():
                def upd(_):
                    v = work_v[pl.ds(base, LANES)]
                    g2l = plsc.load_gather(work_v, [il], mask=ml)
                    g2r = plsc.load_gather(work_v, [ir], mask=mr)
                    g2l = jnp.where(ml, g2l, 0)
                    g2r = jnp.where(mr, g2r, 0)
                    nv = v | g2l | g2r
                    work_v[pl.ds(base, LANES)] = nv
                    return jnp.any(nv != v)

                lax.while_loop(lambda ch: ch, upd, jnp.bool_(True))

            v = work_v[pl.ds(base, LANES)]
            aux_v[pl.ds(base, LANES)] = v & 1
            buf2_v[pl.ds(base, LANES)] = (v >> 1) & 1
            return carry

        lax.fori_loop(0, NCH, sweep, 0)
        pltpu.sync_copy(aux_v, score_out)
        pltpu.sync_copy(buf2_v, reroll_out)


@jax.jit
def _dag_flags(left, right, reroll_i32):
    mesh = plsc.VectorSubcoreMesh(core_axis_name="c", subcore_axis_name="s")
    f = pl.kernel(
        _sc_body,
        out_type=(
            jax.ShapeDtypeStruct((N,), jnp.int32),
            jax.ShapeDtypeStruct((N,), jnp.int32),
            jax.ShapeDtypeStruct((N,), jnp.int32),
            jax.ShapeDtypeStruct((LANES,), jnp.int32),
        ),
        mesh=mesh,
        compiler_params=pltpu.CompilerParams(needs_layout_passes=False),
        scratch_types=(
            pltpu.VMEM((N,), jnp.int32),
            pltpu.VMEM((N,), jnp.int32),
            pltpu.VMEM((N,), jnp.int32),
            pltpu.VMEM((N,), jnp.int32),
            pltpu.VMEM((N,), jnp.int32),
            pltpu.VMEM((N,), jnp.int32),
            pltpu.VMEM((LANES,), jnp.int32),
        ),
    )
    return f(left, right, reroll_i32)


def kernel(thresholds, rules, binary_ops, left, right, leaf_is_reroll,
           leaf_mask_left, leaf_mask_right, leaf_mask_op, leaf_score_cat):
    mask_i, score_i, reroll_i, cnt = _dag_flags(
        left, right, leaf_is_reroll.astype(jnp.int32))
    return (mask_i.astype(jnp.bool_), score_i.astype(jnp.bool_),
            reroll_i.astype(jnp.bool_), cnt[0])


# speedup vs baseline: 16987.1839x; 1.0333x over previous
import functools

import jax
import jax.numpy as jnp
from jax import lax
from jax.experimental import pallas as pl
from jax.experimental.pallas import tpu as pltpu
from jax.experimental.pallas import tpu_sc as plsc

N = 8192
LANES = 16
NCH = N // LANES


def _sc_body(left_hbm, right_hbm, reroll_hbm,
             mask_out, score_out, reroll_out, cnt_out,
             left_v, right_v, aux_v, buf2_v, work_v, cnt_v):
    c = lax.axis_index("c")
    s = lax.axis_index("s")

    @pl.when((s == 0) & (c == 0))
    def _reachability():
        pltpu.sync_copy(left_hbm, left_v)
        pltpu.sync_copy(right_hbm, right_v)

        def zero_chunk(i, carry):
            work_v[pl.ds(i * LANES, LANES)] = jnp.zeros((LANES,), jnp.int32)
            return carry

        lax.fori_loop(0, NCH, zero_chunk, 0)
        onehot0 = (lax.broadcasted_iota(jnp.int32, (LANES,), 0) == 0)
        work_v[pl.ds(0, LANES)] = onehot0.astype(jnp.int32)

        ones = jnp.ones((LANES,), jnp.int32)

        def one_pass(carry):
            _, cur = carry

            def chunk(i, acc):
                base = i * LANES
                v = work_v[pl.ds(base, LANES)]
                lv = left_v[pl.ds(base, LANES)]
                rv = right_v[pl.ds(base, LANES)]
                reach = v != 0
                ml = reach & (lv >= 0)
                mr = reach & (rv >= 0)
                il = jnp.where(ml, lv, 0)
                ir = jnp.where(mr, rv, 0)
                plsc.store_scatter(work_v, [il], ones, mask=ml)
                plsc.store_scatter(work_v, [ir], ones, mask=mr)
                return acc + v

            acc = lax.fori_loop(0, NCH, chunk, jnp.zeros((LANES,), jnp.int32))
            return (cur, jnp.sum(acc))

        _, final_cnt = lax.while_loop(
            lambda pc: pc[0] != pc[1], one_pass,
            (jnp.int32(-1), jnp.int32(0)))

        cnt_v[...] = jnp.full((LANES,), final_cnt, jnp.int32)
        pltpu.sync_copy(work_v.at[pl.ds(0, N)], mask_out)
        pltpu.sync_copy(cnt_v, cnt_out)

    @pl.when((s == 1) & (c == 0))
    def _leaf_flags():
        pltpu.sync_copy(left_hbm, left_v)
        pltpu.sync_copy(right_hbm, right_v)
        pltpu.sync_copy(reroll_hbm, work_v.at[pl.ds(N, N)])

        def sweep(t, carry):
            i = NCH - 1 - t
            base = i * LANES
            nid = base + lax.broadcasted_iota(jnp.int32, (LANES,), 0)
            lv = left_v[pl.ds(base, LANES)]
            rv = right_v[pl.ds(base, LANES)]
            work_v[pl.ds(base, LANES)] = jnp.zeros((LANES,), jnp.int32)

            ml = (lv < 0) | (lv > nid)
            mr = (rv < 0) | (rv > nid)
            il = jnp.where(ml, jnp.where(lv < 0, (N - 1) - lv, lv), 0)
            ir = jnp.where(mr, jnp.where(rv < 0, (N - 1) - rv, rv), 0)

            def contrib():
                gl = plsc.load_gather(work_v, [il], mask=ml)
                gr = plsc.load_gather(work_v, [ir], mask=mr)
                gl = jnp.where(ml, gl, 0)
                gr = jnp.where(mr, gr, 0)
                cl = jnp.where(lv < 0, gl + 1, gl)
                cr = jnp.where(rv < 0, gr + 1, gr)
                return cl | cr

            work_v[pl.ds(base, LANES)] = contrib()

            inchunk = ((lv > nid) & (lv < base + LANES)) | \
                      ((rv > nid) & (rv < base + LANES))

            @pl.when(jnp.any(inchunk))
            def _fixpoint():
                def upd(_):
                    v = work_v[pl.ds(base, LANES)]
                    nv = contrib()
                    work_v[pl.ds(base, LANES)] = nv
                    return jnp.any(nv != v)

                lax.while_loop(lambda ch: ch, upd, jnp.bool_(True))

            v = work_v[pl.ds(base, LANES)]
            aux_v[pl.ds(base, LANES)] = v & 1
            buf2_v[pl.ds(base, LANES)] = (v >> 1) & 1
            return carry

        lax.fori_loop(0, NCH, sweep, 0)
        pltpu.sync_copy(aux_v, score_out)
        pltpu.sync_copy(buf2_v, reroll_out)


@jax.jit
def _dag_flags(left, right, reroll_i32):
    mesh = plsc.VectorSubcoreMesh(core_axis_name="c", subcore_axis_name="s",
                                  num_cores=1)
    f = pl.kernel(
        _sc_body,
        out_type=(
            jax.ShapeDtypeStruct((N,), jnp.int32),
            jax.ShapeDtypeStruct((N,), jnp.int32),
            jax.ShapeDtypeStruct((N,), jnp.int32),
            jax.ShapeDtypeStruct((LANES,), jnp.int32),
        ),
        mesh=mesh,
        compiler_params=pltpu.CompilerParams(needs_layout_passes=False),
        scratch_types=(
            pltpu.VMEM((N,), jnp.int32),
            pltpu.VMEM((N,), jnp.int32),
            pltpu.VMEM((N,), jnp.int32),
            pltpu.VMEM((N,), jnp.int32),
            pltpu.VMEM((2 * N,), jnp.int32),
            pltpu.VMEM((LANES,), jnp.int32),
        ),
    )
    return f(left, right, reroll_i32)


def kernel(thresholds, rules, binary_ops, left, right, leaf_is_reroll,
           leaf_mask_left, leaf_mask_right, leaf_mask_op, leaf_score_cat):
    mask_i, score_i, reroll_i, cnt = _dag_flags(
        left, right, leaf_is_reroll.astype(jnp.int32))
    return (mask_i.astype(jnp.bool_), score_i.astype(jnp.bool_),
            reroll_i.astype(jnp.bool_), cnt[0])
